# baseline (speedup 1.0000x reference)
import jax
import jax.numpy as jnp
import numpy as np
from jax.experimental import pallas as pl
from jax.experimental.pallas import tpu as pltpu

_ANCHORS = np.array(
    [[0.57273, 0.677385], [1.87446, 2.06253], [3.33843, 5.47434],
     [7.88282, 3.52778], [9.77052, 9.16828]], dtype=np.float32)

_A = 5
_CH = 85
_NCLS = _CH - 5


def _softplus(x):
    return jnp.maximum(x, 0.0) + jnp.log1p(jnp.exp(-jnp.abs(x)))


def _loss_body(*refs):
    o_ref = refs[-1]
    Ps = []
    Ts = []
    for a in range(_A):
        blk = refs[a][...]
        cb = blk.shape[0] * blk.shape[1] * blk.shape[2]
        Ps.append(jnp.transpose(blk.reshape(cb, _CH), (1, 0)))
    for a in range(_A):
        blk = refs[_A + a][...]
        cb = blk.shape[0] * blk.shape[1] * blk.shape[2]
        Ts.append(jnp.transpose(blk.reshape(cb, _CH), (1, 0)))

    def chrow(mats, c):
        return jnp.concatenate([m[c:c + 1, :] for m in mats], axis=0)

    aw = jnp.concatenate(
        [jnp.full((1, 1), float(v), jnp.float32) for v in _ANCHORS[:, 0]], axis=0)
    ah = jnp.concatenate(
        [jnp.full((1, 1), float(v), jnp.float32) for v in _ANCHORS[:, 1]], axis=0)

    p_tx = chrow(Ps, 0)
    p_ty = chrow(Ps, 1)
    p_tw = chrow(Ps, 2)
    p_th = chrow(Ps, 3)
    p_to = chrow(Ps, 4)
    t_tx = chrow(Ts, 0)
    t_ty = chrow(Ts, 1)
    t_tw = chrow(Ts, 2)
    t_th = chrow(Ts, 3)
    t_to = chrow(Ts, 4)

    psx = jax.nn.sigmoid(p_tx)
    psy = jax.nn.sigmoid(p_ty)
    pw = jnp.exp(p_tw) * aw
    ph = jnp.exp(p_th) * ah
    gw = jnp.exp(t_tw) * aw
    gh = jnp.exp(t_th) * ah

    px1 = psx - pw * 0.5
    px2 = psx + pw * 0.5
    py1 = psy - ph * 0.5
    py2 = psy + ph * 0.5
    parea = (px2 - px1) * (py2 - py1)

    gx1 = t_tx - gw * 0.5
    gx2 = t_tx + gw * 0.5
    gy1 = t_ty - gh * 0.5
    gy2 = t_ty + gh * 0.5
    garea = (gx2 - gx1) * (gy2 - gy1)

    takenf = jnp.zeros(p_tx.shape, dtype=jnp.float32)
    updfs = []
    for g in range(_A):
        ix1 = jnp.maximum(px1, gx1[g:g + 1, :])
        iy1 = jnp.maximum(py1, gy1[g:g + 1, :])
        ix2 = jnp.minimum(px2, gx2[g:g + 1, :])
        iy2 = jnp.minimum(py2, gy2[g:g + 1, :])
        iw = jnp.maximum(ix2 - ix1, 0.0)
        ih = jnp.maximum(iy2 - iy1, 0.0)
        inter = iw * ih
        union = parea + garea[g:g + 1, :] - inter + 1e-09
        iou = inter / union
        iou = jnp.where(takenf > 0.5, -1.0, iou)
        m = jnp.max(iou, axis=0, keepdims=True)
        e = (iou == m).astype(jnp.float32)
        seen = e[0:1, :]
        rows = [seen]
        for a in range(1, _A):
            rows.append(e[a:a + 1, :] * (1.0 - seen))
            seen = jnp.maximum(seen, e[a:a + 1, :])
        oh = jnp.concatenate(rows, axis=0)
        isobj = (t_to[g:g + 1, :] > 0.5).astype(jnp.float32)
        upd = oh * isobj
        takenf = takenf + upd
        updfs.append(upd)

    objf = takenf

    def gather_tgt(tch):
        s = updfs[0] * tch[0:1, :]
        for g in range(1, _A):
            s = s + updfs[g] * tch[g:g + 1, :]
        return s

    alx = gather_tgt(t_tx)
    aly = gather_tgt(t_ty)
    alw = gather_tgt(t_tw)
    alh = gather_tgt(t_th)

    sxy = jnp.sum(objf * ((psx - alx) ** 2 + (psy - aly) ** 2))
    swh = jnp.sum(objf * ((p_tw - alw) ** 2 + (p_th - alh) ** 2))
    sobj = jnp.sum(objf * _softplus(-p_to))
    snoobj = jnp.sum((1.0 - objf) * _softplus(p_to))

    iota = jax.lax.broadcasted_iota(
        jnp.int32, (_NCLS, p_tx.shape[1]), 0).astype(jnp.float32)
    gidx = []
    for g in range(_A):
        tc = Ts[g][5:_CH, :]
        am = jnp.max(tc, axis=0, keepdims=True)
        gidx.append(jnp.min(jnp.where(tc == am, iota, float(_NCLS)),
                            axis=0, keepdims=True))
    cls_acc = None
    for a in range(_A):
        pc = Ps[a][5:_CH, :]
        m = jnp.max(pc, axis=0, keepdims=True)
        se = jnp.sum(jnp.exp(pc - m), axis=0, keepdims=True)
        lse = m + jnp.log(se)
        idx = updfs[0][a:a + 1, :] * gidx[0]
        for g in range(1, _A):
            idx = idx + updfs[g][a:a + 1, :] * gidx[g]
        psel = jnp.sum(jnp.where(iota == idx, pc, 0.0), axis=0, keepdims=True)
        term = objf[a:a + 1, :] * (lse - psel)
        cls_acc = term if cls_acc is None else cls_acc + term
    scls = jnp.sum(cls_acc)

    @pl.when(pl.program_id(0) == 0)
    def _init():
        for i in range(5):
            o_ref[i] = 0.0

    o_ref[0] += sxy
    o_ref[1] += swh
    o_ref[2] += sobj
    o_ref[3] += snoobj
    o_ref[4] += scls


def _make_call(N, SS, M, interpret=False):
    specs = []
    for a in range(_A):
        specs.append(pl.BlockSpec(
            (M, 1, N, _CH), lambda i, _a=a: (i, _a, 0, 0)))
    specs = specs + [pl.BlockSpec(
        (M, 1, N, _CH), lambda i, _a=a: (i, _a, 0, 0)) for a in range(_A)]
    return pl.pallas_call(
        _loss_body,
        grid=(SS // M,),
        in_specs=specs,
        out_specs=pl.BlockSpec(memory_space=pltpu.SMEM),
        out_shape=jax.ShapeDtypeStruct((5,), jnp.float32),
        interpret=interpret,
    )


def kernel(pred, target):
    N, S, _, A, ch = pred.shape
    SS = S * S
    pv = jnp.transpose(pred, (1, 2, 3, 0, 4)).reshape(SS, A, N, ch)
    tv = jnp.transpose(target, (1, 2, 3, 0, 4)).reshape(SS, A, N, ch)
    M = 26 if SS % 26 == 0 else SS
    args = [pv] * _A + [tv] * _A
    sums = _make_call(N, SS, M)(*args)
    n = jnp.float32(N)
    lxy = 5.0 * sums[0] / n
    lwh = 5.0 * sums[1] / n
    lobj = 1.0 * sums[2] / n
    lnoobj = 0.5 * sums[3] / n
    lcls = 1.0 * sums[4] / n
    total = lxy + lwh + lobj + lnoobj + lcls
    return (total, lxy, lwh, lobj, lnoobj, lcls)


# speedup vs baseline: 1.1057x; 1.1057x over previous
import jax
import jax.numpy as jnp
import numpy as np
from jax.experimental import pallas as pl
from jax.experimental.pallas import tpu as pltpu

_ANCHORS = np.array(
    [[0.57273, 0.677385], [1.87446, 2.06253], [3.33843, 5.47434],
     [7.88282, 3.52778], [9.77052, 9.16828]], dtype=np.float32)

_A = 5
_CH = 85
_NCLS = _CH - 5


def _softplus(x):
    return jnp.maximum(x, 0.0) + jnp.log1p(jnp.exp(-jnp.abs(x)))


def _loss_body(*refs):
    o_ref = refs[-1]
    Ps = []
    Ts = []
    for a in range(_A):
        blk = refs[a][...]
        cb = blk.shape[0] * blk.shape[1] * blk.shape[2]
        Ps.append(jnp.transpose(blk.reshape(cb, _CH), (1, 0)))
    for a in range(_A):
        blk = refs[_A + a][...]
        cb = blk.shape[0] * blk.shape[1] * blk.shape[2]
        Ts.append(jnp.transpose(blk.reshape(cb, _CH), (1, 0)))

    def chrow(mats, c):
        return jnp.concatenate([m[c:c + 1, :] for m in mats], axis=0)

    aw = jnp.concatenate(
        [jnp.full((1, 1), float(v), jnp.float32) for v in _ANCHORS[:, 0]], axis=0)
    ah = jnp.concatenate(
        [jnp.full((1, 1), float(v), jnp.float32) for v in _ANCHORS[:, 1]], axis=0)

    p_tx = chrow(Ps, 0)
    p_ty = chrow(Ps, 1)
    p_tw = chrow(Ps, 2)
    p_th = chrow(Ps, 3)
    p_to = chrow(Ps, 4)
    t_tx = chrow(Ts, 0)
    t_ty = chrow(Ts, 1)
    t_tw = chrow(Ts, 2)
    t_th = chrow(Ts, 3)
    t_to = chrow(Ts, 4)

    psx = jax.nn.sigmoid(p_tx)
    psy = jax.nn.sigmoid(p_ty)
    pw = jnp.exp(p_tw) * aw
    ph = jnp.exp(p_th) * ah
    gw = jnp.exp(t_tw) * aw
    gh = jnp.exp(t_th) * ah

    px1 = psx - pw * 0.5
    px2 = psx + pw * 0.5
    py1 = psy - ph * 0.5
    py2 = psy + ph * 0.5
    parea = (px2 - px1) * (py2 - py1)

    gx1 = t_tx - gw * 0.5
    gx2 = t_tx + gw * 0.5
    gy1 = t_ty - gh * 0.5
    gy2 = t_ty + gh * 0.5
    garea = (gx2 - gx1) * (gy2 - gy1)

    takenf = jnp.zeros(p_tx.shape, dtype=jnp.float32)
    updfs = []
    for g in range(_A):
        ix1 = jnp.maximum(px1, gx1[g:g + 1, :])
        iy1 = jnp.maximum(py1, gy1[g:g + 1, :])
        ix2 = jnp.minimum(px2, gx2[g:g + 1, :])
        iy2 = jnp.minimum(py2, gy2[g:g + 1, :])
        iw = jnp.maximum(ix2 - ix1, 0.0)
        ih = jnp.maximum(iy2 - iy1, 0.0)
        inter = iw * ih
        union = parea + garea[g:g + 1, :] - inter + 1e-09
        iou = inter / union
        iou = jnp.where(takenf > 0.5, -1.0, iou)
        m = jnp.max(iou, axis=0, keepdims=True)
        e = (iou == m).astype(jnp.float32)
        seen = e[0:1, :]
        rows = [seen]
        for a in range(1, _A):
            rows.append(e[a:a + 1, :] * (1.0 - seen))
            seen = jnp.maximum(seen, e[a:a + 1, :])
        oh = jnp.concatenate(rows, axis=0)
        isobj = (t_to[g:g + 1, :] > 0.5).astype(jnp.float32)
        upd = oh * isobj
        takenf = takenf + upd
        updfs.append(upd)

    objf = takenf

    def gather_tgt(tch):
        s = updfs[0] * tch[0:1, :]
        for g in range(1, _A):
            s = s + updfs[g] * tch[g:g + 1, :]
        return s

    alx = gather_tgt(t_tx)
    aly = gather_tgt(t_ty)
    alw = gather_tgt(t_tw)
    alh = gather_tgt(t_th)

    sxy = jnp.sum(objf * ((psx - alx) ** 2 + (psy - aly) ** 2))
    swh = jnp.sum(objf * ((p_tw - alw) ** 2 + (p_th - alh) ** 2))
    sobj = jnp.sum(objf * _softplus(-p_to))
    snoobj = jnp.sum((1.0 - objf) * _softplus(p_to))

    iota = jax.lax.broadcasted_iota(
        jnp.int32, (_NCLS, p_tx.shape[1]), 0).astype(jnp.float32)
    gidx = []
    for g in range(_A):
        tc = Ts[g][5:_CH, :]
        am = jnp.max(tc, axis=0, keepdims=True)
        gidx.append(jnp.min(jnp.where(tc == am, iota, float(_NCLS)),
                            axis=0, keepdims=True))
    cls_acc = None
    for a in range(_A):
        pc = Ps[a][5:_CH, :]
        se = jnp.sum(jnp.exp(pc), axis=0, keepdims=True)
        lse = jnp.log(se)
        idx = updfs[0][a:a + 1, :] * gidx[0]
        for g in range(1, _A):
            idx = idx + updfs[g][a:a + 1, :] * gidx[g]
        psel = jnp.sum(jnp.where(iota == idx, pc, 0.0), axis=0, keepdims=True)
        term = objf[a:a + 1, :] * (lse - psel)
        cls_acc = term if cls_acc is None else cls_acc + term
    scls = jnp.sum(cls_acc)

    @pl.when(pl.program_id(0) == 0)
    def _init():
        for i in range(5):
            o_ref[i] = 0.0

    o_ref[0] += sxy
    o_ref[1] += swh
    o_ref[2] += sobj
    o_ref[3] += snoobj
    o_ref[4] += scls


def _make_call(N, SS, M, interpret=False):
    specs = []
    for a in range(_A):
        specs.append(pl.BlockSpec(
            (M, 1, N, _CH), lambda i, _a=a: (i, _a, 0, 0)))
    specs = specs + [pl.BlockSpec(
        (M, 1, N, _CH), lambda i, _a=a: (i, _a, 0, 0)) for a in range(_A)]
    return pl.pallas_call(
        _loss_body,
        grid=(SS // M,),
        in_specs=specs,
        out_specs=pl.BlockSpec(memory_space=pltpu.SMEM),
        out_shape=jax.ShapeDtypeStruct((5,), jnp.float32),
        interpret=interpret,
    )


def kernel(pred, target):
    N, S, _, A, ch = pred.shape
    SS = S * S
    pv = jnp.transpose(pred, (1, 2, 3, 0, 4)).reshape(SS, A, N, ch)
    tv = jnp.transpose(target, (1, 2, 3, 0, 4)).reshape(SS, A, N, ch)
    M = 52 if SS % 52 == 0 else SS
    args = [pv] * _A + [tv] * _A
    sums = _make_call(N, SS, M)(*args)
    n = jnp.float32(N)
    lxy = 5.0 * sums[0] / n
    lwh = 5.0 * sums[1] / n
    lobj = 1.0 * sums[2] / n
    lnoobj = 0.5 * sums[3] / n
    lcls = 1.0 * sums[4] / n
    total = lxy + lwh + lobj + lnoobj + lcls
    return (total, lxy, lwh, lobj, lnoobj, lcls)


# speedup vs baseline: 1.3142x; 1.1886x over previous
import functools
import jax
import jax.numpy as jnp
import numpy as np
from jax.experimental import pallas as pl
from jax.experimental.pallas import tpu as pltpu

_ANCHORS = np.array(
    [[0.57273, 0.677385], [1.87446, 2.06253], [3.33843, 5.47434],
     [7.88282, 3.52778], [9.77052, 9.16828]], dtype=np.float32)

_A = 5
_CH = 85
_NCLS = _CH - 5


def _softplus(x):
    return jnp.maximum(x, 0.0) + jnp.log1p(jnp.exp(-jnp.abs(x)))


def _loss_body(inv_n, *refs):
    o_ref = refs[-2]
    acc_ref = refs[-1]
    Ps = []
    Ts = []
    for a in range(_A):
        blk = refs[a][...]
        cb = blk.shape[0] * blk.shape[1] * blk.shape[2]
        Ps.append(jnp.transpose(blk.reshape(cb, _CH), (1, 0)))
    for a in range(_A):
        blk = refs[_A + a][...]
        cb = blk.shape[0] * blk.shape[1] * blk.shape[2]
        Ts.append(jnp.transpose(blk.reshape(cb, _CH), (1, 0)))

    def chrow(mats, c):
        return jnp.concatenate([m[c:c + 1, :] for m in mats], axis=0)

    aw = jnp.concatenate(
        [jnp.full((1, 1), float(v), jnp.float32) for v in _ANCHORS[:, 0]], axis=0)
    ah = jnp.concatenate(
        [jnp.full((1, 1), float(v), jnp.float32) for v in _ANCHORS[:, 1]], axis=0)

    p_tx = chrow(Ps, 0)
    p_ty = chrow(Ps, 1)
    p_tw = chrow(Ps, 2)
    p_th = chrow(Ps, 3)
    p_to = chrow(Ps, 4)
    t_tx = chrow(Ts, 0)
    t_ty = chrow(Ts, 1)
    t_tw = chrow(Ts, 2)
    t_th = chrow(Ts, 3)
    t_to = chrow(Ts, 4)

    psx = jax.nn.sigmoid(p_tx)
    psy = jax.nn.sigmoid(p_ty)
    pw = jnp.exp(p_tw) * aw
    ph = jnp.exp(p_th) * ah
    gw = jnp.exp(t_tw) * aw
    gh = jnp.exp(t_th) * ah

    px1 = psx - pw * 0.5
    px2 = psx + pw * 0.5
    py1 = psy - ph * 0.5
    py2 = psy + ph * 0.5
    parea = (px2 - px1) * (py2 - py1)

    gx1 = t_tx - gw * 0.5
    gx2 = t_tx + gw * 0.5
    gy1 = t_ty - gh * 0.5
    gy2 = t_ty + gh * 0.5
    garea = (gx2 - gx1) * (gy2 - gy1)

    takenf = jnp.zeros(p_tx.shape, dtype=jnp.float32)
    updfs = []
    for g in range(_A):
        ix1 = jnp.maximum(px1, gx1[g:g + 1, :])
        iy1 = jnp.maximum(py1, gy1[g:g + 1, :])
        ix2 = jnp.minimum(px2, gx2[g:g + 1, :])
        iy2 = jnp.minimum(py2, gy2[g:g + 1, :])
        iw = jnp.maximum(ix2 - ix1, 0.0)
        ih = jnp.maximum(iy2 - iy1, 0.0)
        inter = iw * ih
        union = parea + garea[g:g + 1, :] - inter + 1e-09
        iou = inter / union
        iou = jnp.where(takenf > 0.5, -1.0, iou)
        m = jnp.max(iou, axis=0, keepdims=True)
        e = (iou == m).astype(jnp.float32)
        seen = e[0:1, :]
        rows = [seen]
        for a in range(1, _A):
            rows.append(e[a:a + 1, :] * (1.0 - seen))
            seen = jnp.maximum(seen, e[a:a + 1, :])
        oh = jnp.concatenate(rows, axis=0)
        isobj = (t_to[g:g + 1, :] > 0.5).astype(jnp.float32)
        upd = oh * isobj
        takenf = takenf + upd
        updfs.append(upd)

    objf = takenf

    def gather_tgt(tch):
        s = updfs[0] * tch[0:1, :]
        for g in range(1, _A):
            s = s + updfs[g] * tch[g:g + 1, :]
        return s

    alx = gather_tgt(t_tx)
    aly = gather_tgt(t_ty)
    alw = gather_tgt(t_tw)
    alh = gather_tgt(t_th)

    sxy = jnp.sum(objf * ((psx - alx) ** 2 + (psy - aly) ** 2))
    swh = jnp.sum(objf * ((p_tw - alw) ** 2 + (p_th - alh) ** 2))
    sobj = jnp.sum(objf * _softplus(-p_to))
    snoobj = jnp.sum((1.0 - objf) * _softplus(p_to))

    iota = jax.lax.broadcasted_iota(
        jnp.int32, (_NCLS, p_tx.shape[1]), 0).astype(jnp.float32)
    gidx = []
    for g in range(_A):
        tc = Ts[g][5:_CH, :]
        am = jnp.max(tc, axis=0, keepdims=True)
        gidx.append(jnp.min(jnp.where(tc == am, iota, float(_NCLS)),
                            axis=0, keepdims=True))
    cls_acc = None
    for a in range(_A):
        pc = Ps[a][5:_CH, :]
        se = jnp.sum(jnp.exp(pc), axis=0, keepdims=True)
        lse = jnp.log(se)
        idx = updfs[0][a:a + 1, :] * gidx[0]
        for g in range(1, _A):
            idx = idx + updfs[g][a:a + 1, :] * gidx[g]
        psel = jnp.sum(jnp.where(iota == idx, pc, 0.0), axis=0, keepdims=True)
        term = objf[a:a + 1, :] * (lse - psel)
        cls_acc = term if cls_acc is None else cls_acc + term
    scls = jnp.sum(cls_acc)

    @pl.when(pl.program_id(0) == 0)
    def _init():
        for i in range(5):
            acc_ref[i] = 0.0

    acc_ref[0] += sxy
    acc_ref[1] += swh
    acc_ref[2] += sobj
    acc_ref[3] += snoobj
    acc_ref[4] += scls

    @pl.when(pl.program_id(0) == pl.num_programs(0) - 1)
    def _fin():
        lxy = 5.0 * acc_ref[0] * inv_n
        lwh = 5.0 * acc_ref[1] * inv_n
        lobj = 1.0 * acc_ref[2] * inv_n
        lnoobj = 0.5 * acc_ref[3] * inv_n
        lcls = 1.0 * acc_ref[4] * inv_n
        o_ref[0] = lxy + lwh + lobj + lnoobj + lcls
        o_ref[1] = lxy
        o_ref[2] = lwh
        o_ref[3] = lobj
        o_ref[4] = lnoobj
        o_ref[5] = lcls


def _make_call(N, SS, M, interpret=False):
    specs = []
    for a in range(_A):
        specs.append(pl.BlockSpec(
            (M, 1, N, _CH), lambda i, _a=a: (i, _a, 0, 0)))
    specs = specs + [pl.BlockSpec(
        (M, 1, N, _CH), lambda i, _a=a: (i, _a, 0, 0)) for a in range(_A)]
    return pl.pallas_call(
        functools.partial(_loss_body, 1.0 / float(N)),
        grid=(SS // M,),
        in_specs=specs,
        out_specs=pl.BlockSpec(memory_space=pltpu.SMEM),
        out_shape=jax.ShapeDtypeStruct((6,), jnp.float32),
        scratch_shapes=[pltpu.SMEM((5,), jnp.float32)],
        interpret=interpret,
    )


def kernel(pred, target):
    N, S, _, A, ch = pred.shape
    SS = S * S
    pv = jnp.transpose(pred, (1, 2, 3, 0, 4)).reshape(SS, A, N, ch)
    tv = jnp.transpose(target, (1, 2, 3, 0, 4)).reshape(SS, A, N, ch)
    M = 52 if SS % 52 == 0 else SS
    args = [pv] * _A + [tv] * _A
    out = _make_call(N, SS, M)(*args)
    return (out[0], out[1], out[2], out[3], out[4], out[5])


# speedup vs baseline: 1.3162x; 1.0015x over previous
import functools
import jax
import jax.numpy as jnp
import numpy as np
from jax.experimental import pallas as pl
from jax.experimental.pallas import tpu as pltpu

_ANCHORS = np.array(
    [[0.57273, 0.677385], [1.87446, 2.06253], [3.33843, 5.47434],
     [7.88282, 3.52778], [9.77052, 9.16828]], dtype=np.float32)

_A = 5
_CH = 85
_NCLS = _CH - 5


def _softplus(x):
    return jnp.maximum(x, 0.0) + jnp.log1p(jnp.exp(-jnp.abs(x)))


def _loss_body(inv_n, *refs):
    o_ref = refs[-2]
    acc_ref = refs[-1]
    Ps = []
    Ts = []
    for a in range(_A):
        blk = refs[a][...]
        cb = blk.shape[0] * blk.shape[1] * blk.shape[2]
        Ps.append(jnp.transpose(blk.reshape(cb, _CH), (1, 0)))
    for a in range(_A):
        blk = refs[_A + a][...]
        cb = blk.shape[0] * blk.shape[1] * blk.shape[2]
        Ts.append(jnp.transpose(blk.reshape(cb, _CH), (1, 0)))

    def chrow(mats, c):
        return jnp.concatenate([m[c:c + 1, :] for m in mats], axis=0)

    aw = jnp.concatenate(
        [jnp.full((1, 1), float(v), jnp.float32) for v in _ANCHORS[:, 0]], axis=0)
    ah = jnp.concatenate(
        [jnp.full((1, 1), float(v), jnp.float32) for v in _ANCHORS[:, 1]], axis=0)

    p_tx = chrow(Ps, 0)
    p_ty = chrow(Ps, 1)
    p_tw = chrow(Ps, 2)
    p_th = chrow(Ps, 3)
    p_to = chrow(Ps, 4)
    t_tx = chrow(Ts, 0)
    t_ty = chrow(Ts, 1)
    t_tw = chrow(Ts, 2)
    t_th = chrow(Ts, 3)
    t_to = chrow(Ts, 4)

    psx = jax.nn.sigmoid(p_tx)
    psy = jax.nn.sigmoid(p_ty)
    pw = jnp.exp(p_tw) * aw
    ph = jnp.exp(p_th) * ah
    gw = jnp.exp(t_tw) * aw
    gh = jnp.exp(t_th) * ah

    px1 = psx - pw * 0.5
    px2 = psx + pw * 0.5
    py1 = psy - ph * 0.5
    py2 = psy + ph * 0.5
    parea = (px2 - px1) * (py2 - py1)

    gx1 = t_tx - gw * 0.5
    gx2 = t_tx + gw * 0.5
    gy1 = t_ty - gh * 0.5
    gy2 = t_ty + gh * 0.5
    garea = (gx2 - gx1) * (gy2 - gy1)

    takenf = jnp.zeros(p_tx.shape, dtype=jnp.float32)
    updfs = []
    for g in range(_A):
        ix1 = jnp.maximum(px1, gx1[g:g + 1, :])
        iy1 = jnp.maximum(py1, gy1[g:g + 1, :])
        ix2 = jnp.minimum(px2, gx2[g:g + 1, :])
        iy2 = jnp.minimum(py2, gy2[g:g + 1, :])
        iw = jnp.maximum(ix2 - ix1, 0.0)
        ih = jnp.maximum(iy2 - iy1, 0.0)
        inter = iw * ih
        union = parea + garea[g:g + 1, :] - inter + 1e-09
        iou = inter / union
        if g > 0:
            iou = jnp.where(takenf > 0.5, -1.0, iou)
        m = jnp.max(iou, axis=0, keepdims=True)
        e = (iou == m).astype(jnp.float32)
        seen = e[0:1, :]
        rows = [seen]
        for a in range(1, _A):
            rows.append(e[a:a + 1, :] * (1.0 - seen))
            seen = jnp.maximum(seen, e[a:a + 1, :])
        oh = jnp.concatenate(rows, axis=0)
        isobj = (t_to[g:g + 1, :] > 0.5).astype(jnp.float32)
        upd = oh * isobj
        takenf = takenf + upd
        updfs.append(upd)

    objf = takenf

    def gather_tgt(tch):
        s = updfs[0] * tch[0:1, :]
        for g in range(1, _A):
            s = s + updfs[g] * tch[g:g + 1, :]
        return s

    alx = gather_tgt(t_tx)
    aly = gather_tgt(t_ty)
    alw = gather_tgt(t_tw)
    alh = gather_tgt(t_th)

    sxy = jnp.sum(objf * ((psx - alx) ** 2 + (psy - aly) ** 2))
    swh = jnp.sum(objf * ((p_tw - alw) ** 2 + (p_th - alh) ** 2))
    sobj = jnp.sum(objf * _softplus(-p_to))
    snoobj = jnp.sum((1.0 - objf) * _softplus(p_to))

    iota = jax.lax.broadcasted_iota(
        jnp.int32, (_NCLS, p_tx.shape[1]), 0).astype(jnp.float32)
    gidx = []
    for g in range(_A):
        tc = Ts[g][5:_CH, :]
        am = jnp.max(tc, axis=0, keepdims=True)
        gidx.append(jnp.min(jnp.where(tc == am, iota, float(_NCLS)),
                            axis=0, keepdims=True))
    idxmat = updfs[0] * gidx[0]
    for g in range(1, _A):
        idxmat = idxmat + updfs[g] * gidx[g]
    cls_acc = None
    for a in range(_A):
        pc = Ps[a][5:_CH, :]
        se = jnp.sum(jnp.exp(pc), axis=0, keepdims=True)
        lse = jnp.log(se)
        idx = idxmat[a:a + 1, :]
        psel = jnp.sum(jnp.where(iota == idx, pc, 0.0), axis=0, keepdims=True)
        term = objf[a:a + 1, :] * (lse - psel)
        cls_acc = term if cls_acc is None else cls_acc + term
    scls = jnp.sum(cls_acc)

    @pl.when(pl.program_id(0) == 0)
    def _init():
        for i in range(5):
            acc_ref[i] = 0.0

    acc_ref[0] += sxy
    acc_ref[1] += swh
    acc_ref[2] += sobj
    acc_ref[3] += snoobj
    acc_ref[4] += scls

    @pl.when(pl.program_id(0) == pl.num_programs(0) - 1)
    def _fin():
        lxy = 5.0 * acc_ref[0] * inv_n
        lwh = 5.0 * acc_ref[1] * inv_n
        lobj = 1.0 * acc_ref[2] * inv_n
        lnoobj = 0.5 * acc_ref[3] * inv_n
        lcls = 1.0 * acc_ref[4] * inv_n
        o_ref[0] = lxy + lwh + lobj + lnoobj + lcls
        o_ref[1] = lxy
        o_ref[2] = lwh
        o_ref[3] = lobj
        o_ref[4] = lnoobj
        o_ref[5] = lcls


def _make_call(N, SS, M, interpret=False):
    specs = []
    for a in range(_A):
        specs.append(pl.BlockSpec(
            (M, 1, N, _CH), lambda i, _a=a: (i, _a, 0, 0)))
    specs = specs + [pl.BlockSpec(
        (M, 1, N, _CH), lambda i, _a=a: (i, _a, 0, 0)) for a in range(_A)]
    return pl.pallas_call(
        functools.partial(_loss_body, 1.0 / float(N)),
        grid=(SS // M,),
        in_specs=specs,
        out_specs=pl.BlockSpec(memory_space=pltpu.SMEM),
        out_shape=jax.ShapeDtypeStruct((6,), jnp.float32),
        scratch_shapes=[pltpu.SMEM((5,), jnp.float32)],
        interpret=interpret,
    )


def kernel(pred, target):
    N, S, _, A, ch = pred.shape
    SS = S * S
    pv = jnp.transpose(pred, (1, 2, 3, 0, 4)).reshape(SS, A, N, ch)
    tv = jnp.transpose(target, (1, 2, 3, 0, 4)).reshape(SS, A, N, ch)
    M = 52 if SS % 52 == 0 else SS
    args = [pv] * _A + [tv] * _A
    out = _make_call(N, SS, M)(*args)
    return (out[0], out[1], out[2], out[3], out[4], out[5])
